# Initial kernel scaffold; baseline (speedup 1.0000x reference)
import jax
import jax.numpy as jnp
from jax.experimental import pallas as pl


def kernel(input, target):
    raise NotImplementedError("write your pallas kernel here")


# speedup vs baseline: 28.7105x; 28.7105x over previous
import functools

import jax
import jax.numpy as jnp
from jax.experimental import pallas as pl


def _body(nsteps, inv_b, t_ref, x_ref, o_ref):
    z = x_ref[...]
    r = z.shape[0]
    t = t_ref[0, 0, :].reshape(r, 1)

    m = jnp.max(z, axis=-1, keepdims=True)

    col = jax.lax.broadcasted_iota(jnp.int32, z.shape, 1)
    zk = jnp.sum(jnp.where(col == t, z, 0.0), axis=-1)

    lo = m - 1.0
    hi = m
    for _ in range(8):
        mid = 0.5 * (lo + hi)
        s = jnp.sum(jnp.maximum(z - mid, 0.0), axis=-1, keepdims=True)
        gt = s > 1.0
        lo = jnp.where(gt, mid, lo)
        hi = jnp.where(gt, hi, mid)

    tau = lo
    for _ in range(8):
        mask = z > tau
        cnt = jnp.sum(mask.astype(jnp.float32), axis=-1, keepdims=True)
        s = jnp.sum(jnp.where(mask, z, 0.0), axis=-1, keepdims=True)
        tau = jnp.maximum(tau, (s - 1.0) / cnt)

    d = z - m
    ts = tau - m
    z_t2 = jnp.sum(jnp.maximum(d * d - ts * ts, 0.0), axis=-1)
    partial = jnp.sum(0.5 * (z_t2 + 1.0) - zk).reshape(1, 1)

    i = pl.program_id(0)

    @pl.when(i == 0)
    def _():
        o_ref[...] = jnp.zeros_like(o_ref)

    o_ref[...] += partial

    @pl.when(i == nsteps - 1)
    def _():
        o_ref[...] = o_ref[...] * inv_b


def kernel(input, target):
    b, c = input.shape
    r = 8
    g = b // r
    t3 = target.astype(jnp.int32).reshape(g, 1, r)
    out = pl.pallas_call(
        functools.partial(_body, g, 1.0 / b),
        grid=(g,),
        in_specs=[
            pl.BlockSpec((1, 1, r), lambda i: (i, 0, 0)),
            pl.BlockSpec((r, c), lambda i: (i, 0)),
        ],
        out_specs=pl.BlockSpec((1, 1), lambda i: (0, 0)),
        out_shape=jax.ShapeDtypeStruct((1, 1), jnp.float32),
    )(t3, input)
    return out[0, 0]


# speedup vs baseline: 54.1817x; 1.8872x over previous
import functools

import jax
import jax.numpy as jnp
from jax.experimental import pallas as pl


def _body(nsteps, inv_b, t_ref, x_ref, o_ref):
    z = x_ref[...]
    r = z.shape[0]
    t = t_ref[0, 0, :].reshape(r, 1)

    m = jnp.max(z, axis=-1, keepdims=True)

    col = jax.lax.broadcasted_iota(jnp.int32, z.shape, 1)
    zk = jnp.sum(jnp.where(col == t, z, 0.0), axis=-1)

    lo = m - 1.0
    hi = m
    for _ in range(6):
        mid = 0.5 * (lo + hi)
        s = jnp.sum(jnp.maximum(z - mid, 0.0), axis=-1, keepdims=True)
        gt = s > 1.0
        lo = jnp.where(gt, mid, lo)
        hi = jnp.where(gt, hi, mid)

    tau = lo
    for _ in range(4):
        mask = z > tau
        cnt = jnp.sum(mask.astype(jnp.float32), axis=-1, keepdims=True)
        s = jnp.sum(jnp.where(mask, z, 0.0), axis=-1, keepdims=True)
        tau = jnp.maximum(tau, (s - 1.0) / cnt)

    d = z - m
    ts = tau - m
    z_t2 = jnp.sum(jnp.maximum(d * d - ts * ts, 0.0), axis=-1)
    partial = jnp.sum(0.5 * (z_t2 + 1.0) - zk).reshape(1, 1)

    i = pl.program_id(0)

    @pl.when(i == 0)
    def _():
        o_ref[...] = jnp.zeros_like(o_ref)

    o_ref[...] += partial

    @pl.when(i == nsteps - 1)
    def _():
        o_ref[...] = o_ref[...] * inv_b


def kernel(input, target):
    b, c = input.shape
    r = 16
    g = b // r
    t3 = target.astype(jnp.int32).reshape(g, 1, r)
    out = pl.pallas_call(
        functools.partial(_body, g, 1.0 / b),
        grid=(g,),
        in_specs=[
            pl.BlockSpec((1, 1, r), lambda i: (i, 0, 0)),
            pl.BlockSpec((r, c), lambda i: (i, 0)),
        ],
        out_specs=pl.BlockSpec((1, 1), lambda i: (0, 0)),
        out_shape=jax.ShapeDtypeStruct((1, 1), jnp.float32),
    )(t3, input)
    return out[0, 0]


# speedup vs baseline: 60.0828x; 1.1089x over previous
import functools

import jax
import jax.numpy as jnp
from jax.experimental import pallas as pl


def _body(nsteps, inv_b, t_ref, x_ref, o_ref):
    z = x_ref[...]
    r = z.shape[0]
    t = t_ref[0, 0, :].reshape(r, 1)

    m = jnp.max(z, axis=-1, keepdims=True)

    col = jax.lax.broadcasted_iota(jnp.int32, z.shape, 1)
    zk = jnp.sum(jnp.where(col == t, z, 0.0), axis=-1)

    lo = m - 1.0
    hi = m
    for _ in range(6):
        mid = 0.5 * (lo + hi)
        s = jnp.sum(jnp.maximum(z - mid, 0.0), axis=-1, keepdims=True)
        gt = s > 1.0
        lo = jnp.where(gt, mid, lo)
        hi = jnp.where(gt, hi, mid)

    tau = lo
    for _ in range(4):
        mask = z > tau
        cnt = jnp.sum(mask.astype(jnp.float32), axis=-1, keepdims=True)
        s = jnp.sum(jnp.where(mask, z, 0.0), axis=-1, keepdims=True)
        tau = jnp.maximum(tau, (s - 1.0) / cnt)

    d = z - m
    ts = tau - m
    z_t2 = jnp.sum(jnp.maximum(d * d - ts * ts, 0.0), axis=-1)
    partial = jnp.sum(0.5 * (z_t2 + 1.0) - zk).reshape(1, 1)

    i = pl.program_id(0)

    @pl.when(i == 0)
    def _():
        o_ref[...] = jnp.zeros_like(o_ref)

    o_ref[...] += partial

    @pl.when(i == nsteps - 1)
    def _():
        o_ref[...] = o_ref[...] * inv_b


def kernel(input, target):
    b, c = input.shape
    r = 32
    g = b // r
    t3 = target.astype(jnp.int32).reshape(g, 1, r)
    out = pl.pallas_call(
        functools.partial(_body, g, 1.0 / b),
        grid=(g,),
        in_specs=[
            pl.BlockSpec((1, 1, r), lambda i: (i, 0, 0)),
            pl.BlockSpec((r, c), lambda i: (i, 0)),
        ],
        out_specs=pl.BlockSpec((1, 1), lambda i: (0, 0)),
        out_shape=jax.ShapeDtypeStruct((1, 1), jnp.float32),
    )(t3, input)
    return out[0, 0]
